# baseline (speedup 1.0000x reference)
import functools

import jax
import jax.numpy as jnp
from jax import lax
from jax.experimental import pallas as pl
from jax.experimental.pallas import tpu as pltpu
from jax.experimental.pallas import tpu_sc as plsc

VOCAB = 1000000
EMB = 64
MAX_NORM = 1.0
BATCH = 16384

NC = 2
NS = 16
L = 16
NW = NC * NS
BPW = BATCH // NW
CB = 32
NCHUNK = BPW // CB
NGROUP = BPW // L


def _rsqrt(s):
    i = plsc.bitcast(s, jnp.int32)
    y = plsc.bitcast(jnp.int32(0x5F3759DF) - (i >> 1), jnp.float32)
    for _ in range(3):
        y = y * (1.5 - 0.5 * s * y * y)
    return y


def _sg_body(iidx_hbm, oidx_hbm, win_hbm, wout_hbm, o_hbm,
             iidx_v, oidx_v, rin_v, rout_v, res_v, sem):
    wid = lax.axis_index("s") * NC + lax.axis_index("c")
    base = wid * BPW

    pltpu.sync_copy(iidx_hbm.at[wid], iidx_v)
    pltpu.sync_copy(oidx_hbm.at[wid], oidx_v)

    lanes = lax.iota(jnp.int32, L)

    def issue(c, _):
        for g in range(CB // L):
            j0 = c * CB + g * L
            ivec = iidx_v[pl.ds(j0, L)]
            ovec = oidx_v[pl.ds(j0, L)]
            for j in range(L):
                sel = lanes == j
                t_in = jnp.max(jnp.where(sel, ivec, 0))
                t_out = jnp.max(jnp.where(sel, ovec, 0))
                pltpu.async_copy(
                    win_hbm.at[pl.ds(t_in, 1)],
                    rin_v.at[pl.ds(j0 + j, 1)], sem)
                pltpu.async_copy(
                    wout_hbm.at[pl.ds(t_out, 1)],
                    rout_v.at[pl.ds(j0 + j, 1)], sem)
        return _

    lax.fori_loop(0, NCHUNK, issue, None)
    pltpu.make_async_copy(win_hbm.at[pl.ds(0, BPW)], rin_v, sem).wait()
    pltpu.make_async_copy(wout_hbm.at[pl.ds(0, BPW)], rout_v, sem).wait()

    def group(g, _):
        elem = g * L + lanes
        s_in = jnp.zeros((L,), jnp.float32)
        s_out = jnp.zeros((L,), jnp.float32)
        dot = jnp.zeros((L,), jnp.float32)
        for e in range(EMB):
            col = jnp.full((L,), e, jnp.int32)
            a = plsc.load_gather(rin_v, [elem, col])
            b = plsc.load_gather(rout_v, [elem, col])
            s_in = s_in + a * a
            s_out = s_out + b * b
            dot = dot + a * b
        scale = jnp.minimum(1.0, MAX_NORM * _rsqrt(s_in)) * \
            jnp.minimum(1.0, MAX_NORM * _rsqrt(s_out))
        x = dot * scale
        res_v[pl.ds(g * L, L)] = 1.0 / (1.0 + jnp.exp(-x))
        return _

    lax.fori_loop(0, NGROUP, group, None)
    pltpu.sync_copy(res_v, o_hbm.at[pl.ds(base, BPW)])


@jax.jit
def _skipgram(iidx, oidx, w_in, w_out):
    run = functools.partial(
        pl.kernel,
        mesh=plsc.VectorSubcoreMesh(core_axis_name="c", subcore_axis_name="s"),
        out_type=jax.ShapeDtypeStruct((BATCH,), jnp.float32),
        scratch_types=[
            pltpu.VMEM((BPW,), jnp.int32),
            pltpu.VMEM((BPW,), jnp.int32),
            pltpu.VMEM((BPW, EMB), jnp.float32),
            pltpu.VMEM((BPW, EMB), jnp.float32),
            pltpu.VMEM((BPW,), jnp.float32),
            pltpu.SemaphoreType.DMA,
        ],
        compiler_params=pltpu.CompilerParams(
            needs_layout_passes=False, use_tc_tiling_on_sc=False),
    )(_sg_body)
    return run(iidx, oidx, w_in, w_out)


def kernel(inputs, outputs, W_in, W_out):
    iidx = inputs.reshape(NW, BPW).astype(jnp.int32)
    oidx = outputs.reshape(NW, BPW).astype(jnp.int32)
    return _skipgram(iidx, oidx, W_in, W_out)


# speedup vs baseline: 1.5214x; 1.5214x over previous
import functools

import jax
import jax.numpy as jnp
from jax import lax
from jax.experimental import pallas as pl
from jax.experimental.pallas import tpu as pltpu
from jax.experimental.pallas import tpu_sc as plsc

VOCAB = 1000000
EMB = 64
MAX_NORM = 1.0
BATCH = 16384

NC = 2
NS = 16
L = 16
NW = NC * NS
BPW = BATCH // NW
CB = 32
NCHUNK = BPW // CB
GPC = CB // L


def _rsqrt(s):
    i = plsc.bitcast(s, jnp.int32)
    y = plsc.bitcast(jnp.int32(0x5F3759DF) - (i >> 1), jnp.float32)
    for _ in range(3):
        y = y * (1.5 - 0.5 * s * y * y)
    return y


def _sg_body(iidx_hbm, oidx_hbm, win_hbm, wout_hbm, o_hbm,
             iidx_v, oidx_v, rin_v, rout_v, res_v, sem):
    wid = lax.axis_index("s") * NC + lax.axis_index("c")
    base = wid * BPW

    pltpu.sync_copy(iidx_hbm.at[wid], iidx_v)
    pltpu.sync_copy(oidx_hbm.at[wid], oidx_v)

    lanes = lax.iota(jnp.int32, L)

    def chunk(c, _):
        copies = []
        for g in range(GPC):
            ivec = iidx_v[pl.ds(c * CB + g * L, L)]
            ovec = oidx_v[pl.ds(c * CB + g * L, L)]
            for j in range(L):
                sel = lanes == j
                t_in = jnp.max(jnp.where(sel, ivec, 0))
                t_out = jnp.max(jnp.where(sel, ovec, 0))
                copies.append(pltpu.async_copy(
                    win_hbm.at[pl.ds(t_in, 1)],
                    rin_v.at[pl.ds(g * L + j, 1)], sem))
                copies.append(pltpu.async_copy(
                    wout_hbm.at[pl.ds(t_out, 1)],
                    rout_v.at[pl.ds(g * L + j, 1)], sem))
        for cp in copies:
            cp.wait()
        for g in range(GPC):
            elem = g * L + lanes
            s_in = jnp.zeros((L,), jnp.float32)
            s_out = jnp.zeros((L,), jnp.float32)
            dot = jnp.zeros((L,), jnp.float32)
            for e in range(EMB):
                col = jnp.full((L,), e, jnp.int32)
                a = plsc.load_gather(rin_v, [elem, col])
                b = plsc.load_gather(rout_v, [elem, col])
                s_in = s_in + a * a
                s_out = s_out + b * b
                dot = dot + a * b
            scale = jnp.minimum(1.0, MAX_NORM * _rsqrt(s_in)) * \
                jnp.minimum(1.0, MAX_NORM * _rsqrt(s_out))
            x = dot * scale
            res_v[pl.ds(c * CB + g * L, L)] = 1.0 / (1.0 + jnp.exp(-x))
        return _

    lax.fori_loop(0, NCHUNK, chunk, None)
    pltpu.sync_copy(res_v, o_hbm.at[pl.ds(base, BPW)])


@jax.jit
def _skipgram(iidx, oidx, w_in, w_out):
    run = functools.partial(
        pl.kernel,
        mesh=plsc.VectorSubcoreMesh(core_axis_name="c", subcore_axis_name="s"),
        out_type=jax.ShapeDtypeStruct((BATCH,), jnp.float32),
        scratch_types=[
            pltpu.VMEM((BPW,), jnp.int32),
            pltpu.VMEM((BPW,), jnp.int32),
            pltpu.VMEM((CB, EMB), jnp.float32),
            pltpu.VMEM((CB, EMB), jnp.float32),
            pltpu.VMEM((BPW,), jnp.float32),
            pltpu.SemaphoreType.DMA,
        ],
        compiler_params=pltpu.CompilerParams(needs_layout_passes=False),
    )(_sg_body)
    return run(iidx, oidx, w_in, w_out)


def kernel(inputs, outputs, W_in, W_out):
    iidx = inputs.reshape(NW, BPW).astype(jnp.int32)
    oidx = outputs.reshape(NW, BPW).astype(jnp.int32)
    return _skipgram(iidx, oidx, W_in, W_out)


# speedup vs baseline: 2.4830x; 1.6320x over previous
import functools

import jax
import jax.numpy as jnp
from jax import lax
from jax.experimental import pallas as pl
from jax.experimental.pallas import tpu as pltpu
from jax.experimental.pallas import tpu_sc as plsc

VOCAB = 1000000
EMB = 64
MAX_NORM = 1.0
BATCH = 16384

NC = 2
NS = 16
L = 16
NW = NC * NS
BPW = BATCH // NW
NGROUP = BPW // L
NBLK = (VOCAB + 127) // 128
CAPB = 16
NBUF = 3
MAXB = (NBLK // NW) + 2
KMAX = (MAXB + NBUF - 1) // NBUF
CNTSZ = 288
GCH = 1024
RPAD = 1024
DCH = 64


def _rsqrt(s):
    i = plsc.bitcast(s, jnp.int32)
    y = plsc.bitcast(jnp.int32(0x5F3759DF) - (i >> 1), jnp.float32)
    for _ in range(3):
        y = y * (1.5 - 0.5 * s * y * y)
    return y


def _extract(vec, j):
    return jnp.max(jnp.where(lax.iota(jnp.int32, L) == j, vec, 0))


def _gather_body(iidx_hbm, oidx_hbm, win_hbm, wout_hbm, rows_all,
                 gidx_v, cnt_v, bk_v, bufs, stages, csems, osems):
    wid = lax.axis_index("s") * NC + lax.axis_index("c")
    lanes = lax.iota(jnp.int32, L)
    bs = (wid * NBLK + NW - 1) // NW
    be = ((wid + 1) * NBLK + NW - 1) // NW
    nblk = be - bs

    for t in range(2):
        idx_hbm = (iidx_hbm, oidx_hbm)[t]
        w_hbm = (win_hbm, wout_hbm)[t]
        rows_hbm = rows_all
        tof = t * BATCH

        for z in range(CNTSZ // L):
            cnt_v[pl.ds(z * L, L)] = jnp.zeros((L,), jnp.int32)

        def scan_chunk(sc, carry):
            pltpu.sync_copy(idx_hbm.at[pl.ds(sc * GCH, GCH)], gidx_v)
            lax.fori_loop(sc * (GCH // L), (sc + 1) * (GCH // L), scan, None)
            return carry

        def scan(g, carry):
            vec = gidx_v[pl.ds((g % (GCH // L)) * L, L)]
            blk = vec >> 7
            own = (blk * NW) // NBLK
            m = own == wid
            tb = jnp.clip(blk - bs, 0, MAXB - 1)
            cur = plsc.load_gather(cnt_v, [tb], mask=m)
            dup, _ = plsc.scan_count(tb, m)
            order = jnp.minimum(cur + dup - 1, CAPB - 1)
            packed = ((g * L + lanes) << 7) | (vec & 127)
            plsc.store_scatter(bk_v, [tb * CAPB + order], packed, mask=m)
            plsc.addupdate_scatter(
                cnt_v, [tb], jnp.ones((L,), jnp.int32), mask=m)
            return carry

        lax.fori_loop(0, BATCH // GCH, scan_chunk, None)

        def fire(b, p):
            off = pl.multiple_of((bs + b) * 128, 128)
            return pltpu.async_copy(
                w_hbm.at[:, pl.ds(off, 128)], bufs[p], csems[p])

        for p in range(NBUF):
            @pl.when(p < nblk)
            def _():
                fire(p, p)

        def block_iter(k, carry):
            newc = carry
            for p in range(NBUF):
                b = k * NBUF + p

                @pl.when(b < nblk)
                def _():
                    pltpu.make_async_copy(
                        w_hbm.at[:, pl.ds(0, 128)], bufs[p], csems[p]).wait()

                cvec = cnt_v[pl.ds(b, L)]
                c = jnp.minimum(_extract(cvec, 0), CAPB)

                @pl.when((b < nblk) & (c > 0))
                def _():
                    pn = _extract(newc, p)

                    def drain(j, cr):
                        pltpu.make_async_copy(
                            rows_hbm.at[pl.ds(0, 1)],
                            stages[p].at[pl.ds(0, 1)], osems[p]).wait()
                        return cr
                    lax.fori_loop(0, pn, drain, None)

                    pk = bk_v[pl.ds(b * CAPB, L)]
                    em = lanes < c
                    pos = pk >> 7
                    col = pk & 127
                    for e in range(EMB):
                        ev = jnp.full((L,), e, jnp.int32)
                        val = plsc.load_gather(bufs[p], [ev, col], mask=em)
                        plsc.store_scatter(
                            stages[p], [lanes, ev], val, mask=em)

                    def put(j, cr):
                        pj = _extract(pos, j) + tof
                        pltpu.async_copy(
                            stages[p].at[pl.ds(j, 1)],
                            rows_hbm.at[pl.ds(pj, 1)], osems[p])
                        return cr
                    lax.fori_loop(0, c, put, None)

                nb = b + NBUF

                @pl.when(nb < nblk)
                def _():
                    fire(nb, p)

                newc = jnp.where((lanes == p) & (b < nblk) & (c > 0),
                                 c, newc)
            return newc

        counts = lax.fori_loop(0, KMAX, block_iter,
                               jnp.zeros((L,), jnp.int32))

        for p in range(NBUF):
            pn = _extract(counts, p)

            def drain(j, cr):
                pltpu.make_async_copy(
                    rows_hbm.at[pl.ds(0, 1)],
                    stages[p].at[pl.ds(0, 1)], osems[p]).wait()
                return cr
            lax.fori_loop(0, pn, drain, None)


def _dot_body(rows_all, o_hbm, rin_v, rout_v, res_v, sem):
    wid = lax.axis_index("s") * NC + lax.axis_index("c")
    base = wid * BPW
    lanes = lax.iota(jnp.int32, L)

    def chunk(c, _):
        cb = base + c * DCH
        cp_a = pltpu.async_copy(rows_all.at[pl.ds(cb, DCH)], rin_v, sem)
        cp_b = pltpu.async_copy(
            rows_all.at[pl.ds(BATCH + cb, DCH)], rout_v, sem)
        cp_a.wait()
        cp_b.wait()
        for g in range(DCH // L):
            elem = g * L + lanes
            s_in = jnp.zeros((L,), jnp.float32)
            s_out = jnp.zeros((L,), jnp.float32)
            dot = jnp.zeros((L,), jnp.float32)
            for e in range(EMB):
                col = jnp.full((L,), e, jnp.int32)
                a = plsc.load_gather(rin_v, [elem, col])
                b = plsc.load_gather(rout_v, [elem, col])
                s_in = s_in + a * a
                s_out = s_out + b * b
                dot = dot + a * b
            scale = jnp.minimum(1.0, MAX_NORM * _rsqrt(s_in)) * \
                jnp.minimum(1.0, MAX_NORM * _rsqrt(s_out))
            x = dot * scale
            res_v[pl.ds(c * DCH + g * L, L)] = 1.0 / (1.0 + jnp.exp(-x))
        return _

    lax.fori_loop(0, BPW // DCH, chunk, None)
    pltpu.sync_copy(res_v, o_hbm.at[pl.ds(base, BPW)])


@jax.jit
def _skipgram(iidx, oidx, w_in_t, w_out_t):
    mesh = plsc.VectorSubcoreMesh(core_axis_name="c", subcore_axis_name="s")
    gather = functools.partial(
        pl.kernel,
        mesh=mesh,
        out_type=pltpu.MemorySpace.HBM((2 * BATCH + RPAD, EMB), jnp.float32),
        scratch_types={
            "gidx_v": pltpu.VMEM((GCH,), jnp.int32),
            "cnt_v": pltpu.VMEM((CNTSZ,), jnp.int32),
            "bk_v": pltpu.VMEM((MAXB * CAPB + L,), jnp.int32),
            "bufs": [pltpu.VMEM((EMB, 128), jnp.float32)
                     for _ in range(NBUF)],
            "stages": [pltpu.VMEM((CAPB, EMB), jnp.float32)
                       for _ in range(NBUF)],
            "csems": [pltpu.SemaphoreType.DMA for _ in range(NBUF)],
            "osems": [pltpu.SemaphoreType.DMA for _ in range(NBUF)],
        },
        compiler_params=pltpu.CompilerParams(
            needs_layout_passes=False, disable_bounds_checks=True),
    )(_gather_body)
    rows_all = gather(iidx, oidx, w_in_t, w_out_t)

    dot = functools.partial(
        pl.kernel,
        mesh=mesh,
        out_type=jax.ShapeDtypeStruct((BATCH,), jnp.float32),
        scratch_types=[
            pltpu.VMEM((DCH, EMB), jnp.float32),
            pltpu.VMEM((DCH, EMB), jnp.float32),
            pltpu.VMEM((BPW,), jnp.float32),
            pltpu.SemaphoreType.DMA,
        ],
        compiler_params=pltpu.CompilerParams(needs_layout_passes=False),
    )(_dot_body)
    rows_all = pltpu.with_memory_space_constraint(
        rows_all, pltpu.MemorySpace.HBM)
    return dot(rows_all)


def kernel(inputs, outputs, W_in, W_out):
    iidx = inputs.reshape(BATCH).astype(jnp.int32)
    oidx = outputs.reshape(BATCH).astype(jnp.int32)
    return _skipgram(iidx, oidx, W_in.T, W_out.T)
